# baseline (device time: 60696 ns/iter reference)
import jax
import jax.numpy as jnp
from jax import lax
from jax.experimental import pallas as pl
from jax.experimental.pallas import tpu as pltpu

C = 32


def kernel(x):
    _, m, n = x.shape
    half = m // 2
    ck = half // C

    def body(x_ref, out_ref, send_buf, rs_recv,
             rs_ss, rs_rs, agy_ss, agy_rs, agx_ss, agx_rs, fwd_ss, fwd_rs):
        my_x = lax.axis_index("x")
        my_y = lax.axis_index("y")
        ox = 1 - my_x
        oy = 1 - my_y
        xp = (ox, my_y)
        yp = (my_x, oy)

        r_me = my_x * half
        r_ot = ox * half
        c_me = my_y * n

        barrier = pltpu.get_barrier_semaphore()
        for nbr in (xp, yp):
            pl.semaphore_signal(
                barrier, inc=1, device_id=nbr,
                device_id_type=pl.DeviceIdType.MESH,
            )
        pl.semaphore_wait(barrier, 2)

        rs = []
        for k in range(C):
            send_buf[pl.ds(k * ck, ck), :] = x_ref[
                0, pl.ds(r_ot + k * ck, ck), :
            ].astype(jnp.bfloat16)
            d = pltpu.make_async_remote_copy(
                src_ref=send_buf.at[pl.ds(k * ck, ck), :],
                dst_ref=rs_recv.at[pl.ds(k * ck, ck), :],
                send_sem=rs_ss.at[k],
                recv_sem=rs_rs.at[k],
                device_id=xp,
                device_id_type=pl.DeviceIdType.MESH,
            )
            d.start()
            rs.append(d)

        ag_y, ag_x = [], []
        for k in range(C):
            rs[k].wait_recv()
            out_ref[pl.ds(r_me + k * ck, ck), pl.ds(c_me, n)] = (
                x_ref[0, pl.ds(r_me + k * ck, ck), :].astype(jnp.bfloat16)
                + rs_recv[pl.ds(k * ck, ck), :]
            )
            src = out_ref.at[pl.ds(r_me + k * ck, ck), pl.ds(c_me, n)]
            dx = pltpu.make_async_remote_copy(
                src_ref=src, dst_ref=src,
                send_sem=agx_ss.at[k], recv_sem=agx_rs.at[k],
                device_id=xp, device_id_type=pl.DeviceIdType.MESH,
            )
            dy = pltpu.make_async_remote_copy(
                src_ref=src, dst_ref=src,
                send_sem=agy_ss.at[k], recv_sem=agy_rs.at[k],
                device_id=yp, device_id_type=pl.DeviceIdType.MESH,
            )
            dx.start()
            dy.start()
            ag_x.append(dx)
            ag_y.append(dy)

        fwd = []
        for k in range(C):
            ag_x[k].wait_recv()
            src = out_ref.at[pl.ds(r_ot + k * ck, ck), pl.ds(c_me, n)]
            d = pltpu.make_async_remote_copy(
                src_ref=src, dst_ref=src,
                send_sem=fwd_ss.at[k], recv_sem=fwd_rs.at[k],
                device_id=yp, device_id_type=pl.DeviceIdType.MESH,
            )
            d.start()
            fwd.append(d)

        for k in range(C):
            rs[k].wait_send()
            ag_x[k].wait_send()
            ag_y[k].wait()
            fwd[k].wait()

    return pl.pallas_call(
        body,
        out_shape=jax.ShapeDtypeStruct((m, 2 * n), jnp.bfloat16),
        in_specs=[pl.BlockSpec(memory_space=pltpu.VMEM)],
        out_specs=pl.BlockSpec(memory_space=pltpu.VMEM),
        scratch_shapes=[
            pltpu.VMEM((half, n), jnp.bfloat16),
            pltpu.VMEM((half, n), jnp.bfloat16),
            pltpu.SemaphoreType.DMA((C,)),
            pltpu.SemaphoreType.DMA((C,)),
            pltpu.SemaphoreType.DMA((C,)),
            pltpu.SemaphoreType.DMA((C,)),
            pltpu.SemaphoreType.DMA((C,)),
            pltpu.SemaphoreType.DMA((C,)),
            pltpu.SemaphoreType.DMA((C,)),
            pltpu.SemaphoreType.DMA((C,)),
        ],
        compiler_params=pltpu.CompilerParams(collective_id=0),
    )(x)
